# Initial kernel scaffold; baseline (speedup 1.0000x reference)
import jax
import jax.numpy as jnp
from jax.experimental import pallas as pl


def kernel(inputs, gate_W, gate_b, W1, b1, W2, b2):
    raise NotImplementedError("write your pallas kernel here")


# speedup vs baseline: 1.2714x; 1.2714x over previous
import functools

import jax
import jax.numpy as jnp
from jax import lax
from jax.experimental import pallas as pl
from jax.experimental.pallas import tpu as pltpu

B, S, D = 2, 2048, 1024
E = 8
K = 2
FF = 4096
T = B * S
NP = T * K
TM = 256
NT = (NP + E * (TM - 1) + TM - 1) // TM
P = NT * TM
BT = 512


def _gate_body(x_ref, gw_ref, gb_ref, ef_ref, rf_ref, w_ref, cnt_ref, carry_ref):
    pid = pl.program_id(0)

    @pl.when(pid == 0)
    def _():
        carry_ref[...] = jnp.zeros_like(carry_ref)

    x = x_ref[...]
    logits = jnp.dot(x, gw_ref[...], preferred_element_type=jnp.float32)
    logits = logits + gb_ref[...]
    eids = lax.broadcasted_iota(jnp.int32, (BT, E), 1)
    m1 = jnp.max(logits, axis=1, keepdims=True)
    a1 = jnp.min(jnp.where(logits == m1, eids, E), axis=1, keepdims=True)
    masked = jnp.where(eids == a1, -1e30, logits)
    m2 = jnp.max(masked, axis=1, keepdims=True)
    a2 = jnp.min(jnp.where(masked == m2, eids, E), axis=1, keepdims=True)
    t = jnp.exp(m2 - m1)
    w1 = 1.0 / (1.0 + t)
    w2 = t * w1

    oh1 = (eids == a1).astype(jnp.float32)
    oh2 = (eids == a2).astype(jnp.float32)
    both = oh1 + oh2
    ti = lax.broadcasted_iota(jnp.int32, (BT, BT), 0)
    tj = lax.broadcasted_iota(jnp.int32, (BT, BT), 1)
    tril = (tj < ti).astype(jnp.float32)
    csum = jnp.dot(tril, both, preferred_element_type=jnp.float32)
    base = csum + carry_ref[...]
    r1 = jnp.sum(oh1 * base, axis=1, keepdims=True)
    r2 = jnp.sum(oh2 * (base + oh1), axis=1, keepdims=True)
    carry_new = carry_ref[...] + jnp.sum(both, axis=0, keepdims=True)
    carry_ref[...] = carry_new

    ef_ref[...] = jnp.concatenate([a1, a2], axis=1)
    rf_ref[...] = jnp.concatenate([r1, r2], axis=1).astype(jnp.int32)
    w_ref[...] = jnp.concatenate([w1, w2], axis=1)
    cnt_ref[...] = carry_new.astype(jnp.int32)


def _gate_call(x, gate_W, gate_b):
    grid = (T // BT,)
    return pl.pallas_call(
        _gate_body,
        grid=grid,
        in_specs=[
            pl.BlockSpec((BT, D), lambda i: (i, 0)),
            pl.BlockSpec((D, E), lambda i: (0, 0)),
            pl.BlockSpec((E,), lambda i: (0,)),
        ],
        out_specs=[
            pl.BlockSpec((BT, K), lambda i: (i, 0)),
            pl.BlockSpec((BT, K), lambda i: (i, 0)),
            pl.BlockSpec((BT, K), lambda i: (i, 0)),
            pl.BlockSpec((1, E), lambda i: (0, 0)),
        ],
        out_shape=[
            jax.ShapeDtypeStruct((T, K), jnp.int32),
            jax.ShapeDtypeStruct((T, K), jnp.int32),
            jax.ShapeDtypeStruct((T, K), jnp.float32),
            jax.ShapeDtypeStruct((1, E), jnp.int32),
        ],
        scratch_shapes=[pltpu.VMEM((1, E), jnp.float32)],
    )(x, gate_W, gate_b)


NFC = 2
FFC = FF // NFC


def _ffn_body(te_ref, xd_ref, w1_ref, b1_ref, w2_ref, b2_ref, y_ref, acc_ref):
    j = pl.program_id(1)
    x = xd_ref[...]
    h = jnp.dot(x, w1_ref[0], preferred_element_type=jnp.float32) + b1_ref[0]
    h = jax.nn.gelu(h)
    part = jnp.dot(h, w2_ref[0], preferred_element_type=jnp.float32)

    @pl.when(j == 0)
    def _():
        acc_ref[...] = part

    @pl.when(j > 0)
    def _():
        acc_ref[...] += part

    @pl.when(j == NFC - 1)
    def _():
        y_ref[...] = acc_ref[...] + b2_ref[0]


def _ffn_call(te, xd, W1, b1, W2, b2):
    grid_spec = pltpu.PrefetchScalarGridSpec(
        num_scalar_prefetch=1,
        grid=(NT, NFC),
        in_specs=[
            pl.BlockSpec((TM, D), lambda i, j, te: (i, 0)),
            pl.BlockSpec((1, D, FFC), lambda i, j, te: (te[i], 0, j)),
            pl.BlockSpec((1, 1, FFC), lambda i, j, te: (te[i], 0, j)),
            pl.BlockSpec((1, FFC, D), lambda i, j, te: (te[i], j, 0)),
            pl.BlockSpec((1, 1, D), lambda i, j, te: (te[i], 0, 0)),
        ],
        out_specs=pl.BlockSpec((TM, D), lambda i, j, te: (i, 0)),
        scratch_shapes=[pltpu.VMEM((TM, D), jnp.float32)],
    )
    return pl.pallas_call(
        _ffn_body,
        grid_spec=grid_spec,
        out_shape=jax.ShapeDtypeStruct((P, D), jnp.float32),
    )(te, xd, W1, b1.reshape(E, 1, FF), W2, b2.reshape(E, 1, D))


def kernel(inputs, gate_W, gate_b, W1, b1, W2, b2):
    x = inputs.reshape(T, D)
    ef, rf, w, counts = _gate_call(x, gate_W, gate_b)
    counts = counts.reshape(E)
    ef_f = ef.reshape(NP)
    rf_f = rf.reshape(NP)

    cp = ((counts + TM - 1) // TM) * TM
    ends = jnp.cumsum(cp)
    offs = ends - cp
    dst = offs[ef_f] + rf_f
    src = jnp.arange(NP, dtype=jnp.int32) // K
    xd = jnp.zeros((P, D), jnp.float32).at[dst].set(x[src])
    te = jnp.clip(
        jnp.searchsorted(ends, jnp.arange(NT, dtype=jnp.int32) * TM, side="right"),
        0, E - 1).astype(jnp.int32)

    y = _ffn_call(te, xd, W1, b1, W2, b2)

    y1 = y[dst[0::2]]
    y2 = y[dst[1::2]]
    out = w[:, 0:1] * y1 + w[:, 1:2] * y2
    return out.reshape(B, S, D)


# speedup vs baseline: 1.4205x; 1.1173x over previous
import functools

import jax
import jax.numpy as jnp
from jax import lax
from jax.experimental import pallas as pl
from jax.experimental.pallas import tpu as pltpu

B, S, D = 2, 2048, 1024
E = 8
K = 2
FF = 4096
T = B * S
NP = T * K
TM = 256
NT = (NP + E * (TM - 1) + TM - 1) // TM
P = NT * TM
BT = 512


def _gate_body(x_ref, gw_ref, gb_ref, ef_ref, rf_ref, w_ref, cnt_ref, carry_ref):
    pid = pl.program_id(0)

    @pl.when(pid == 0)
    def _():
        carry_ref[...] = jnp.zeros_like(carry_ref)

    x = x_ref[...]
    logits = jnp.dot(x, gw_ref[...], preferred_element_type=jnp.float32)
    logits = logits + gb_ref[...]
    eids = lax.broadcasted_iota(jnp.int32, (BT, E), 1)
    m1 = jnp.max(logits, axis=1, keepdims=True)
    a1 = jnp.min(jnp.where(logits == m1, eids, E), axis=1, keepdims=True)
    masked = jnp.where(eids == a1, -1e30, logits)
    m2 = jnp.max(masked, axis=1, keepdims=True)
    a2 = jnp.min(jnp.where(masked == m2, eids, E), axis=1, keepdims=True)
    t = jnp.exp(m2 - m1)
    w1 = 1.0 / (1.0 + t)
    w2 = t * w1

    oh1 = (eids == a1).astype(jnp.float32)
    oh2 = (eids == a2).astype(jnp.float32)
    both = oh1 + oh2
    ti = lax.broadcasted_iota(jnp.int32, (BT, BT), 0)
    tj = lax.broadcasted_iota(jnp.int32, (BT, BT), 1)
    tril = (tj < ti).astype(jnp.float32)
    csum = jnp.dot(tril, both, preferred_element_type=jnp.float32)
    base = csum + carry_ref[...]
    r1 = jnp.sum(oh1 * base, axis=1, keepdims=True)
    r2 = jnp.sum(oh2 * (base + oh1), axis=1, keepdims=True)
    carry_new = carry_ref[...] + jnp.sum(both, axis=0, keepdims=True)
    carry_ref[...] = carry_new

    ef_ref[...] = jnp.concatenate([a1, a2], axis=1)
    rf_ref[...] = jnp.concatenate([r1, r2], axis=1).astype(jnp.int32)
    w_ref[...] = jnp.concatenate([w1, w2], axis=1)
    cnt_ref[...] = carry_new.astype(jnp.int32)


def _gate_call(x, gate_W, gate_b):
    grid = (T // BT,)
    return pl.pallas_call(
        _gate_body,
        grid=grid,
        in_specs=[
            pl.BlockSpec((BT, D), lambda i: (i, 0)),
            pl.BlockSpec((D, E), lambda i: (0, 0)),
            pl.BlockSpec((E,), lambda i: (0,)),
        ],
        out_specs=[
            pl.BlockSpec((BT, K), lambda i: (i, 0)),
            pl.BlockSpec((BT, K), lambda i: (i, 0)),
            pl.BlockSpec((BT, K), lambda i: (i, 0)),
            pl.BlockSpec((1, E), lambda i: (0, 0)),
        ],
        out_shape=[
            jax.ShapeDtypeStruct((T, K), jnp.int32),
            jax.ShapeDtypeStruct((T, K), jnp.int32),
            jax.ShapeDtypeStruct((T, K), jnp.float32),
            jax.ShapeDtypeStruct((1, E), jnp.int32),
        ],
        scratch_shapes=[pltpu.VMEM((1, E), jnp.float32)],
    )(x, gate_W, gate_b)


def _ffn_body(te_ref, xd_ref, w1_ref, b1_ref, w2_ref, b2_ref, y_ref):
    x = xd_ref[...].astype(jnp.bfloat16)
    h = jnp.dot(x, w1_ref[0], preferred_element_type=jnp.float32) + b1_ref[0]
    h = jax.nn.gelu(h).astype(jnp.bfloat16)
    y_ref[...] = jnp.dot(h, w2_ref[0], preferred_element_type=jnp.float32) + b2_ref[0]


def _ffn_call(te, xd, W1, b1, W2, b2):
    grid_spec = pltpu.PrefetchScalarGridSpec(
        num_scalar_prefetch=1,
        grid=(NT,),
        in_specs=[
            pl.BlockSpec((TM, D), lambda i, te: (i, 0)),
            pl.BlockSpec((1, D, FF), lambda i, te: (te[i], 0, 0)),
            pl.BlockSpec((1, 1, FF), lambda i, te: (te[i], 0, 0)),
            pl.BlockSpec((1, FF, D), lambda i, te: (te[i], 0, 0)),
            pl.BlockSpec((1, 1, D), lambda i, te: (te[i], 0, 0)),
        ],
        out_specs=pl.BlockSpec((TM, D), lambda i, te: (i, 0)),
    )
    return pl.pallas_call(
        _ffn_body,
        grid_spec=grid_spec,
        out_shape=jax.ShapeDtypeStruct((P, D), jnp.float32),
    )(te, xd, W1.astype(jnp.bfloat16), b1.reshape(E, 1, FF),
      W2.astype(jnp.bfloat16), b2.reshape(E, 1, D))


def kernel(inputs, gate_W, gate_b, W1, b1, W2, b2):
    x = inputs.reshape(T, D)
    ef, rf, w, counts = _gate_call(x, gate_W, gate_b)
    counts = counts.reshape(E)
    ef_f = ef.reshape(NP)
    rf_f = rf.reshape(NP)

    cp = ((counts + TM - 1) // TM) * TM
    ends = jnp.cumsum(cp)
    offs = ends - cp
    dst = offs[ef_f] + rf_f
    src = jnp.arange(NP, dtype=jnp.int32) // K
    xd = jnp.zeros((P, D), jnp.float32).at[dst].set(x[src])
    te = jnp.clip(
        jnp.searchsorted(ends, jnp.arange(NT, dtype=jnp.int32) * TM, side="right"),
        0, E - 1).astype(jnp.int32)

    y = _ffn_call(te, xd, W1, b1, W2, b2)

    y1 = y[dst[0::2]]
    y2 = y[dst[1::2]]
    out = w[:, 0:1] * y1 + w[:, 1:2] * y2
    return out.reshape(B, S, D)


# speedup vs baseline: 1.4999x; 1.0559x over previous
import functools

import jax
import jax.numpy as jnp
from jax import lax
from jax.experimental import pallas as pl
from jax.experimental.pallas import tpu as pltpu

B, S, D = 2, 2048, 1024
E = 8
K = 2
FF = 4096
T = B * S
NP = T * K
TM = 256
NT = (NP + E * (TM - 1) + TM - 1) // TM
P = NT * TM
BT = 512


def _gate_body(x_ref, gw_ref, gb_ref, ef_ref, rf_ref, w_ref, cnt_ref, carry_ref):
    pid = pl.program_id(0)

    @pl.when(pid == 0)
    def _():
        carry_ref[...] = jnp.zeros_like(carry_ref)

    x = x_ref[...]
    logits = jnp.dot(x, gw_ref[...], preferred_element_type=jnp.float32)
    logits = logits + gb_ref[...]
    eids = lax.broadcasted_iota(jnp.int32, (BT, E), 1)
    m1 = jnp.max(logits, axis=1, keepdims=True)
    a1 = jnp.min(jnp.where(logits == m1, eids, E), axis=1, keepdims=True)
    masked = jnp.where(eids == a1, -1e30, logits)
    m2 = jnp.max(masked, axis=1, keepdims=True)
    a2 = jnp.min(jnp.where(masked == m2, eids, E), axis=1, keepdims=True)
    t = jnp.exp(m2 - m1)
    w1 = 1.0 / (1.0 + t)
    w2 = t * w1

    oh1 = (eids == a1).astype(jnp.float32)
    oh2 = (eids == a2).astype(jnp.float32)
    both = oh1 + oh2
    ti = lax.broadcasted_iota(jnp.int32, (BT, BT), 0)
    tj = lax.broadcasted_iota(jnp.int32, (BT, BT), 1)
    tril = (tj < ti).astype(jnp.float32)
    csum = jnp.dot(tril, both, preferred_element_type=jnp.float32)
    base = csum + carry_ref[...]
    r1 = jnp.sum(oh1 * base, axis=1, keepdims=True)
    r2 = jnp.sum(oh2 * (base + oh1), axis=1, keepdims=True)
    carry_new = carry_ref[...] + jnp.sum(both, axis=0, keepdims=True)
    carry_ref[...] = carry_new

    ef_ref[...] = jnp.concatenate([a1, a2], axis=1)
    rf_ref[...] = jnp.concatenate([r1, r2], axis=1).astype(jnp.int32)
    w_ref[...] = jnp.concatenate([w1, w2], axis=1)
    cnt_ref[...] = carry_new.astype(jnp.int32)


def _gate_call(x, gate_W, gate_b):
    grid = (T // BT,)
    return pl.pallas_call(
        _gate_body,
        grid=grid,
        in_specs=[
            pl.BlockSpec((BT, D), lambda i: (i, 0)),
            pl.BlockSpec((D, E), lambda i: (0, 0)),
            pl.BlockSpec((E,), lambda i: (0,)),
        ],
        out_specs=[
            pl.BlockSpec((BT, K), lambda i: (i, 0)),
            pl.BlockSpec((BT, K), lambda i: (i, 0)),
            pl.BlockSpec((BT, K), lambda i: (i, 0)),
            pl.BlockSpec((1, E), lambda i: (0, 0)),
        ],
        out_shape=[
            jax.ShapeDtypeStruct((T, K), jnp.int32),
            jax.ShapeDtypeStruct((T, K), jnp.int32),
            jax.ShapeDtypeStruct((T, K), jnp.float32),
            jax.ShapeDtypeStruct((1, E), jnp.int32),
        ],
        scratch_shapes=[pltpu.VMEM((1, E), jnp.float32)],
    )(x, gate_W, gate_b)


def _ffn_body(te_ref, xd_ref, w1_ref, b1_ref, w2_ref, b2_ref, y_ref):
    x = xd_ref[...].astype(jnp.bfloat16)
    h = jnp.dot(x, w1_ref[0], preferred_element_type=jnp.float32) + b1_ref[0]
    h = jax.nn.gelu(h).astype(jnp.bfloat16)
    y_ref[...] = jnp.dot(h, w2_ref[0], preferred_element_type=jnp.float32) + b2_ref[0]


def _ffn_call(te, xd, W1, b1, W2, b2):
    grid_spec = pltpu.PrefetchScalarGridSpec(
        num_scalar_prefetch=1,
        grid=(NT,),
        in_specs=[
            pl.BlockSpec((TM, D), lambda i, te: (i, 0)),
            pl.BlockSpec((1, D, FF), lambda i, te: (te[i], 0, 0)),
            pl.BlockSpec((1, 1, FF), lambda i, te: (te[i], 0, 0)),
            pl.BlockSpec((1, FF, D), lambda i, te: (te[i], 0, 0)),
            pl.BlockSpec((1, 1, D), lambda i, te: (te[i], 0, 0)),
        ],
        out_specs=pl.BlockSpec((TM, D), lambda i, te: (i, 0)),
    )
    return pl.pallas_call(
        _ffn_body,
        grid_spec=grid_spec,
        out_shape=jax.ShapeDtypeStruct((P, D), jnp.float32),
    )(te, xd, W1.astype(jnp.bfloat16), b1.reshape(E, 1, FF),
      W2.astype(jnp.bfloat16), b2.reshape(E, 1, D))


def kernel(inputs, gate_W, gate_b, W1, b1, W2, b2):
    x = inputs.reshape(T, D)
    ef, rf, w, counts = _gate_call(x, gate_W, gate_b)
    counts = counts.reshape(E)
    ef_f = ef.reshape(NP)
    rf_f = rf.reshape(NP)

    cp = ((counts + TM - 1) // TM) * TM
    ends = jnp.cumsum(cp)
    offs = ends - cp
    dst = offs[ef_f] + rf_f
    src = jnp.arange(NP, dtype=jnp.int32) // K
    srcrow = jnp.zeros((P,), jnp.int32).at[dst].set(src)
    xd = x[srcrow]
    te = jnp.clip(
        jnp.searchsorted(ends, jnp.arange(NT, dtype=jnp.int32) * TM, side="right"),
        0, E - 1).astype(jnp.int32)

    y = _ffn_call(te, xd, W1, b1, W2, b2)

    y1 = y[dst[0::2]]
    y2 = y[dst[1::2]]
    out = w[:, 0:1] * y1 + w[:, 1:2] * y2
    return out.reshape(B, S, D)
